# Initial kernel scaffold; baseline (speedup 1.0000x reference)
import jax
import jax.numpy as jnp
from jax.experimental import pallas as pl


def kernel(trees, indexes, mask_padding, W_emb, gate, W1, b1, W2, b2):
    raise NotImplementedError("write your pallas kernel here")


# speedup vs baseline: 9.8551x; 9.8551x over previous
import jax
import jax.numpy as jnp
from jax.experimental import pallas as pl


def _body(trees_ref, idx_ref, valid_ref, W_ref, gate_ref, W1_ref, b1_ref,
          W2_ref, b2_ref, out_ref, comb_ref, *, BP, F, N):
    f32 = jnp.float32
    W = W_ref[...]
    T_wide = jnp.concatenate([trees_ref[p] for p in range(BP)], axis=1)
    A_wide = jnp.dot(W, T_wide, preferred_element_type=f32)

    row_iota = jax.lax.broadcasted_iota(jnp.int32, (N, N), 0)
    col_iota = jax.lax.broadcasted_iota(jnp.int32, (N, N), 1)
    eye = (row_iota == col_iota).astype(f32)
    onehot1 = (jax.lax.broadcasted_iota(jnp.int32, (1, N), 1) == 1).astype(f32)
    gate_row = gate_ref[...]

    for p in range(BP):
        A = A_wide[:, p * N:(p + 1) * N]
        idx_row = idx_ref[p:p + 1, :]
        E2 = (row_iota == idx_row).astype(f32) + eye
        emb = jnp.maximum(jnp.dot(A, E2, preferred_element_type=f32), 0.0)
        scores = jnp.dot(gate_row, emb, preferred_element_type=f32)
        v = valid_ref[p:p + 1, :]
        scores = jnp.where(v > 0.5, scores, -1e30)
        m = jnp.max(scores, axis=1, keepdims=True)
        e = jnp.exp(scores - m) * v
        denom = jnp.sum(e, axis=1, keepdims=True)
        w_row = e / denom
        S = jnp.concatenate([onehot1, w_row], axis=0)
        R = jax.lax.dot_general(S, emb, (((1,), (1,)), ((), ())),
                                preferred_element_type=f32)
        comb_ref[p:p + 1, 0:F] = R[0:1, :]
        comb_ref[p:p + 1, F:2 * F] = R[1:2, :]

    comb = comb_ref[...]
    h = jnp.maximum(
        jnp.dot(comb, W1_ref[...], preferred_element_type=f32) + b1_ref[...],
        0.0)
    out_ref[...] = jnp.dot(h, W2_ref[...], preferred_element_type=f32) + b2_ref[...]


def kernel(trees, indexes, mask_padding, W_emb, gate, W1, b1, W2, b2):
    P, F, N = trees.shape
    H = W1.shape[1]
    BP = 8
    f32 = jnp.float32

    valid = 1.0 - mask_padding.astype(f32)
    gate2 = gate.reshape(1, F)
    b1_2 = b1.reshape(1, H)
    b2_2 = b2.reshape(1, 1)

    import functools
    body = functools.partial(_body, BP=BP, F=F, N=N)

    out, comb = pl.pallas_call(
        body,
        grid=(P // BP,),
        in_specs=[
            pl.BlockSpec((BP, F, N), lambda i: (i, 0, 0)),
            pl.BlockSpec((BP, N), lambda i: (i, 0)),
            pl.BlockSpec((BP, N), lambda i: (i, 0)),
            pl.BlockSpec((F, F), lambda i: (0, 0)),
            pl.BlockSpec((1, F), lambda i: (0, 0)),
            pl.BlockSpec((2 * F, H), lambda i: (0, 0)),
            pl.BlockSpec((1, H), lambda i: (0, 0)),
            pl.BlockSpec((H, 1), lambda i: (0, 0)),
            pl.BlockSpec((1, 1), lambda i: (0, 0)),
        ],
        out_specs=[
            pl.BlockSpec((BP, 1), lambda i: (i, 0)),
            pl.BlockSpec((BP, 2 * F), lambda i: (i, 0)),
        ],
        out_shape=[
            jax.ShapeDtypeStruct((P, 1), f32),
            jax.ShapeDtypeStruct((P, 2 * F), f32),
        ],
    )(trees, indexes, valid, W_emb, gate2, W1, b1_2, W2, b2_2)
    return (out, comb)


# speedup vs baseline: 16.2983x; 1.6538x over previous
import functools

import jax
import jax.numpy as jnp
from jax.experimental import pallas as pl


def _body(trees_ref, idx_ref, valid_ref, W_ref, gate_ref, W1_ref, b1_ref,
          W2_ref, b2_ref, out_ref, comb_ref, *, BP, F, N):
    f32 = jnp.float32
    bf16 = jnp.bfloat16
    NW = BP * N

    T_wide = jnp.concatenate([trees_ref[p] for p in range(BP)], axis=1)
    A_wide = jnp.dot(W_ref[...], T_wide.astype(bf16),
                     preferred_element_type=f32).astype(bf16)

    row_iota = jax.lax.broadcasted_iota(jnp.int32, (N, N), 0)
    col_iota = jax.lax.broadcasted_iota(jnp.int32, (N, N), 1)
    eye = (row_iota == col_iota).astype(bf16)

    embs = []
    for p in range(BP):
        A = A_wide[:, p * N:(p + 1) * N]
        idx_row = idx_ref[p:p + 1, :]
        E2 = (row_iota == idx_row).astype(bf16) + eye
        embs.append(jnp.maximum(
            jnp.dot(A, E2, preferred_element_type=f32), 0).astype(bf16))
    emb_wide = jnp.concatenate(embs, axis=1)

    scores = jnp.dot(gate_ref[...], emb_wide,
                     preferred_element_type=f32)
    score_mat = jnp.concatenate(
        [scores[:, p * N:(p + 1) * N] for p in range(BP)], axis=0)

    v = valid_ref[...]
    score_mat = jnp.where(v > 0.5, score_mat, -1e30)
    m = jnp.max(score_mat, axis=1, keepdims=True)
    e = jnp.exp(score_mat - m) * v
    denom = jnp.sum(e, axis=1, keepdims=True)
    w_mat = e / denom

    lane = jax.lax.broadcasted_iota(jnp.int32, (BP, NW), 1)
    sub = jax.lax.broadcasted_iota(jnp.int32, (BP, NW), 0)
    root_sel = (lane == sub * N + 1).astype(bf16)
    w_tiled = jnp.concatenate([w_mat] * BP, axis=1)
    w_sel = jnp.where(lane // N == sub, w_tiled, 0.0).astype(bf16)
    SW = jnp.concatenate([root_sel, w_sel], axis=0)

    R = jax.lax.dot_general(SW, emb_wide, (((1,), (1,)), ((), ())),
                            preferred_element_type=f32)
    comb = jnp.concatenate([R[0:BP], R[BP:2 * BP]], axis=1)
    comb_ref[...] = comb

    h = jnp.maximum(
        jnp.dot(comb.astype(bf16), W1_ref[...],
                preferred_element_type=f32) + b1_ref[...], 0.0)
    out_ref[...] = jnp.dot(h.astype(bf16), W2_ref[...],
                           preferred_element_type=f32) + b2_ref[...]


def kernel(trees, indexes, mask_padding, W_emb, gate, W1, b1, W2, b2):
    P, F, N = trees.shape
    H = W1.shape[1]
    BP = 8
    f32 = jnp.float32
    bf16 = jnp.bfloat16

    valid = 1.0 - mask_padding.astype(f32)
    W_bf = W_emb.astype(bf16)
    gate2 = gate.reshape(1, F).astype(bf16)
    W1_bf = W1.astype(bf16)
    W2_bf = W2.astype(bf16)
    b1_2 = b1.reshape(1, H)
    b2_2 = b2.reshape(1, 1)

    body = functools.partial(_body, BP=BP, F=F, N=N)

    out, comb = pl.pallas_call(
        body,
        grid=(P // BP,),
        in_specs=[
            pl.BlockSpec((BP, F, N), lambda i: (i, 0, 0)),
            pl.BlockSpec((BP, N), lambda i: (i, 0)),
            pl.BlockSpec((BP, N), lambda i: (i, 0)),
            pl.BlockSpec((F, F), lambda i: (0, 0)),
            pl.BlockSpec((1, F), lambda i: (0, 0)),
            pl.BlockSpec((2 * F, H), lambda i: (0, 0)),
            pl.BlockSpec((1, H), lambda i: (0, 0)),
            pl.BlockSpec((H, 1), lambda i: (0, 0)),
            pl.BlockSpec((1, 1), lambda i: (0, 0)),
        ],
        out_specs=[
            pl.BlockSpec((BP, 1), lambda i: (i, 0)),
            pl.BlockSpec((BP, 2 * F), lambda i: (i, 0)),
        ],
        out_shape=[
            jax.ShapeDtypeStruct((P, 1), f32),
            jax.ShapeDtypeStruct((P, 2 * F), f32),
        ],
    )(trees, indexes, valid, W_bf, gate2, W1_bf, b1_2, W2_bf, b2_2)
    return (out, comb)


# speedup vs baseline: 28.5813x; 1.7536x over previous
import functools

import jax
import jax.numpy as jnp
from jax.experimental import pallas as pl


def _emb_body(trees_ref, idx_ref, valid_ref, W_ref, gate_ref, riota_ref,
              eye_ref, rsel_ref, bmask_ref, comb_ref, *, BP, F, N):
    f32 = jnp.float32
    bf16 = jnp.bfloat16

    T_wide = jnp.concatenate(
        [trees_ref[p].astype(bf16) for p in range(BP)], axis=1)
    A_wide = jnp.dot(W_ref[...], T_wide,
                     preferred_element_type=f32).astype(bf16)

    riota = riota_ref[...]
    eye = eye_ref[...]
    embs = []
    for p in range(BP):
        A = A_wide[:, p * N:(p + 1) * N]
        idx_bf = idx_ref[p:p + 1, :].astype(bf16)
        E2 = (riota == idx_bf).astype(bf16) + eye
        embs.append(jnp.maximum(
            jnp.dot(A, E2, preferred_element_type=f32).astype(bf16), 0))
    emb_wide = jnp.concatenate(embs, axis=1)

    scores = jnp.dot(gate_ref[...], emb_wide,
                     preferred_element_type=f32)
    score_mat = jnp.concatenate(
        [scores[:, p * N:(p + 1) * N] for p in range(BP)], axis=0)

    v = valid_ref[...]
    score_mat = jnp.where(v > 0.5, score_mat, -1e30)
    m = jnp.max(score_mat, axis=1, keepdims=True)
    e = jnp.exp(score_mat - m) * v
    denom = jnp.sum(e, axis=1, keepdims=True)
    w_bf = (e / denom).astype(bf16)

    w_tiled = jnp.concatenate([w_bf] * BP, axis=1)
    w_sel = w_tiled * bmask_ref[...]
    SW = jnp.concatenate([rsel_ref[...], w_sel], axis=0)

    R = jax.lax.dot_general(SW, emb_wide, (((1,), (1,)), ((), ())),
                            preferred_element_type=f32)
    comb_ref[...] = jnp.concatenate([R[0:BP], R[BP:2 * BP]], axis=1)


def _mlp_body(comb_ref, W1_ref, b1_ref, W2_ref, b2_ref, out_ref):
    f32 = jnp.float32
    bf16 = jnp.bfloat16
    h = jnp.maximum(
        jnp.dot(comb_ref[...].astype(bf16), W1_ref[...],
                preferred_element_type=f32) + b1_ref[...], 0.0)
    out_ref[...] = jnp.dot(h.astype(bf16), W2_ref[...],
                           preferred_element_type=f32) + b2_ref[...]


def kernel(trees, indexes, mask_padding, W_emb, gate, W1, b1, W2, b2):
    P, F, N = trees.shape
    H = W1.shape[1]
    BP = 16
    f32 = jnp.float32
    bf16 = jnp.bfloat16

    valid = 1.0 - mask_padding.astype(f32)
    W_bf = W_emb.astype(bf16)
    gate2 = gate.reshape(1, F).astype(bf16)

    n_iota = jnp.arange(N, dtype=jnp.int32)
    riota = jnp.broadcast_to(n_iota[:, None], (N, N)).astype(bf16)
    eye = jnp.eye(N, dtype=bf16)
    lane = jnp.arange(BP * N, dtype=jnp.int32)[None, :]
    sub = jnp.arange(BP, dtype=jnp.int32)[:, None]
    rsel = (lane == sub * N + 1).astype(bf16)
    bmask = (lane // N == sub).astype(bf16)

    comb = pl.pallas_call(
        functools.partial(_emb_body, BP=BP, F=F, N=N),
        grid=(P // BP,),
        in_specs=[
            pl.BlockSpec((BP, F, N), lambda i: (i, 0, 0)),
            pl.BlockSpec((BP, N), lambda i: (i, 0)),
            pl.BlockSpec((BP, N), lambda i: (i, 0)),
            pl.BlockSpec((F, F), lambda i: (0, 0)),
            pl.BlockSpec((1, F), lambda i: (0, 0)),
            pl.BlockSpec((N, N), lambda i: (0, 0)),
            pl.BlockSpec((N, N), lambda i: (0, 0)),
            pl.BlockSpec((BP, BP * N), lambda i: (0, 0)),
            pl.BlockSpec((BP, BP * N), lambda i: (0, 0)),
        ],
        out_specs=pl.BlockSpec((BP, 2 * F), lambda i: (i, 0)),
        out_shape=jax.ShapeDtypeStruct((P, 2 * F), f32),
    )(trees, indexes, valid, W_bf, gate2, riota, eye, rsel, bmask)

    RB = 512
    out = pl.pallas_call(
        _mlp_body,
        grid=(P // RB,),
        in_specs=[
            pl.BlockSpec((RB, 2 * F), lambda i: (i, 0)),
            pl.BlockSpec((2 * F, H), lambda i: (0, 0)),
            pl.BlockSpec((1, H), lambda i: (0, 0)),
            pl.BlockSpec((H, 1), lambda i: (0, 0)),
            pl.BlockSpec((1, 1), lambda i: (0, 0)),
        ],
        out_specs=pl.BlockSpec((RB, 1), lambda i: (i, 0)),
        out_shape=jax.ShapeDtypeStruct((P, 1), f32),
    )(comb, W1.astype(bf16), b1.reshape(1, H), W2.astype(bf16),
      b2.reshape(1, 1))
    return (out, comb)


# speedup vs baseline: 28.8026x; 1.0077x over previous
import functools

import jax
import jax.numpy as jnp
from jax.experimental import pallas as pl
from jax.experimental.pallas import tpu as pltpu


def _emb_body(trees_ref, idx_ref, valid_ref, W_ref, gate_ref, riota_ref,
              eye_ref, rsel_ref, bmask_ref, comb_ref, emb_buf, *, BP, F, N):
    f32 = jnp.float32
    bf16 = jnp.bfloat16
    i = pl.program_id(0)
    ib = jax.lax.rem(i, 2)

    @pl.when(i > 0)
    def _stage2():
        emb_prev = emb_buf[1 - ib]
        scores = jnp.dot(gate_ref[...], emb_prev,
                         preferred_element_type=f32)
        score_mat = jnp.concatenate(
            [scores[:, p * N:(p + 1) * N] for p in range(BP)], axis=0)
        v = valid_ref[...]
        score_mat = jnp.where(v > 0.5, score_mat, -1e30)
        m = jnp.max(score_mat, axis=1, keepdims=True)
        e = jnp.exp(score_mat - m) * v
        denom = jnp.sum(e, axis=1, keepdims=True)
        w_bf = (e / denom).astype(bf16)

        w_tiled = jnp.concatenate([w_bf] * BP, axis=1)
        w_sel = w_tiled * bmask_ref[...]
        SW = jnp.concatenate([rsel_ref[...], w_sel], axis=0)
        R = jax.lax.dot_general(SW, emb_prev, (((1,), (1,)), ((), ())),
                                preferred_element_type=f32)
        comb_ref[...] = jnp.concatenate([R[0:BP], R[BP:2 * BP]], axis=1)

    W = W_ref[...]
    riota = riota_ref[...]
    eye = eye_ref[...]
    for p in range(BP):
        T_bf = trees_ref[p].astype(bf16)
        A = jnp.dot(W, T_bf, preferred_element_type=f32).astype(bf16)
        idx_bf = idx_ref[p:p + 1, :].astype(bf16)
        E2 = (riota == idx_bf).astype(bf16) + eye
        emb_buf[ib, :, p * N:(p + 1) * N] = jnp.maximum(
            jnp.dot(A, E2, preferred_element_type=f32).astype(bf16), 0)


def _mlp_body(comb_ref, W1_ref, b1_ref, W2_ref, b2_ref, out_ref):
    f32 = jnp.float32
    bf16 = jnp.bfloat16
    h = jnp.maximum(
        jnp.dot(comb_ref[...].astype(bf16), W1_ref[...],
                preferred_element_type=f32) + b1_ref[...], 0.0)
    out_ref[...] = jnp.dot(h.astype(bf16), W2_ref[...],
                           preferred_element_type=f32) + b2_ref[...]


def kernel(trees, indexes, mask_padding, W_emb, gate, W1, b1, W2, b2):
    P, F, N = trees.shape
    H = W1.shape[1]
    BP = 16
    NS = P // BP
    f32 = jnp.float32
    bf16 = jnp.bfloat16

    valid = 1.0 - mask_padding.astype(f32)
    W_bf = W_emb.astype(bf16)
    gate2 = gate.reshape(1, F).astype(bf16)

    n_iota = jnp.arange(N, dtype=jnp.int32)
    riota = jnp.broadcast_to(n_iota[:, None], (N, N)).astype(bf16)
    eye = jnp.eye(N, dtype=bf16)
    lane = jnp.arange(BP * N, dtype=jnp.int32)[None, :]
    sub = jnp.arange(BP, dtype=jnp.int32)[:, None]
    rsel = (lane == sub * N + 1).astype(bf16)
    bmask = (lane // N == sub).astype(bf16)

    comb = pl.pallas_call(
        functools.partial(_emb_body, BP=BP, F=F, N=N),
        grid=(NS + 1,),
        in_specs=[
            pl.BlockSpec((BP, F, N),
                         lambda i: (jnp.minimum(i, NS - 1), 0, 0)),
            pl.BlockSpec((BP, N),
                         lambda i: (jnp.minimum(i, NS - 1), 0)),
            pl.BlockSpec((BP, N),
                         lambda i: (jnp.maximum(i - 1, 0), 0)),
            pl.BlockSpec((F, F), lambda i: (0, 0)),
            pl.BlockSpec((1, F), lambda i: (0, 0)),
            pl.BlockSpec((N, N), lambda i: (0, 0)),
            pl.BlockSpec((N, N), lambda i: (0, 0)),
            pl.BlockSpec((BP, BP * N), lambda i: (0, 0)),
            pl.BlockSpec((BP, BP * N), lambda i: (0, 0)),
        ],
        out_specs=pl.BlockSpec((BP, 2 * F),
                               lambda i: (jnp.maximum(i - 1, 0), 0)),
        out_shape=jax.ShapeDtypeStruct((P, 2 * F), f32),
        scratch_shapes=[pltpu.VMEM((2, F, BP * N), bf16)],
    )(trees, indexes, valid, W_bf, gate2, riota, eye, rsel, bmask)

    RB = 512
    out = pl.pallas_call(
        _mlp_body,
        grid=(P // RB,),
        in_specs=[
            pl.BlockSpec((RB, 2 * F), lambda i: (i, 0)),
            pl.BlockSpec((2 * F, H), lambda i: (0, 0)),
            pl.BlockSpec((1, H), lambda i: (0, 0)),
            pl.BlockSpec((H, 1), lambda i: (0, 0)),
            pl.BlockSpec((1, 1), lambda i: (0, 0)),
        ],
        out_specs=pl.BlockSpec((RB, 1), lambda i: (i, 0)),
        out_shape=jax.ShapeDtypeStruct((P, 1), f32),
    )(comb, W1.astype(bf16), b1.reshape(1, H), W2.astype(bf16),
      b2.reshape(1, 1))
    return (out, comb)
